# Initial kernel scaffold; baseline (speedup 1.0000x reference)
import jax
import jax.numpy as jnp
from jax.experimental import pallas as pl


def kernel(user_features, item_features, user_item_edge_index, item_user_edge_index, Wu, bu, Wi, bi, gamma, beta):
    raise NotImplementedError("write your pallas kernel here")


# speedup vs baseline: 4.6705x; 4.6705x over previous
import functools

import jax
import jax.numpy as jnp
from jax import lax
from jax.experimental import pallas as pl
from jax.experimental.pallas import tpu as pltpu
from jax.experimental.pallas import tpu_sc as plsc

N_NODES = 5000
N_EDGES = 320000
DIM = 128
EPS = 1e-5

NS = 16
PAD = 5120
K = 80
EDGES_PER_TILE = N_EDGES // NS
CHUNKS = EDGES_PER_TILE // K
OUT_STRIDE = 312

_mesh = plsc.VectorSubcoreMesh(core_axis_name="c", subcore_axis_name="s")


@functools.partial(
    pl.kernel,
    out_type=(
        jax.ShapeDtypeStruct((N_NODES, DIM), jnp.float32),
        jax.ShapeDtypeStruct((N_NODES, DIM), jnp.float32),
        jax.ShapeDtypeStruct((N_NODES, DIM), jnp.float32),
        jax.ShapeDtypeStruct((N_NODES, DIM), jnp.float32),
    ),
    mesh=_mesh,
    scratch_types=[
        pltpu.VMEM_SHARED((PAD, DIM), jnp.float32),
        pltpu.VMEM_SHARED((PAD, DIM), jnp.float32),
        pltpu.VMEM((K,), jnp.int32),
        pltpu.VMEM((K,), jnp.int32),
        pltpu.VMEM((K, DIM), jnp.float32),
        pltpu.VMEM((K, DIM), jnp.float32),
        pltpu.SemaphoreType.DMA,
    ],
)
def _sc_aggregate(uf_hbm, if_hbm, ui_src, ui_dst, iu_src, iu_dst,
                  zero_f, ones_hbm,
                  item_sum, item_deg, user_sum, user_deg,
                  acc_sh, deg_sh, src_v, dst_v, rows_v, ones_v, sem):
  cid = lax.axis_index("c")
  sid = lax.axis_index("s")
  row0 = sid * (PAD // NS)
  edge0 = sid * EDGES_PER_TILE

  pltpu.sync_copy(zero_f.at[pl.ds(row0, PAD // NS)],
                  acc_sh.at[pl.ds(row0, PAD // NS)])
  pltpu.sync_copy(zero_f.at[pl.ds(row0, PAD // NS)],
                  deg_sh.at[pl.ds(row0, PAD // NS)])
  pltpu.sync_copy(ones_hbm, ones_v)
  plsc.subcore_barrier()

  def run(feat_hbm, src_hbm, dst_hbm):
    def chunk(c, _):
      off = edge0 + c * K
      pltpu.sync_copy(src_hbm.at[pl.ds(off, K)], src_v)
      pltpu.sync_copy(dst_hbm.at[pl.ds(off, K)], dst_v)
      pltpu.async_copy(feat_hbm.at[src_v], rows_v, sem).wait()
      pltpu.sync_copy(rows_v, acc_sh.at[dst_v], add=True)
      pltpu.sync_copy(ones_v, deg_sh.at[dst_v], add=True)
      return ()
    lax.fori_loop(0, CHUNKS, chunk, ())

  @pl.when(cid == 0)
  def _():
    run(uf_hbm, ui_src, ui_dst)

  @pl.when(cid == 1)
  def _():
    run(if_hbm, iu_src, iu_dst)

  plsc.subcore_barrier()

  out0 = sid * OUT_STRIDE

  @pl.when(cid == 0)
  def _():
    pltpu.sync_copy(acc_sh.at[pl.ds(out0, 320)], item_sum.at[pl.ds(out0, 320)])
    pltpu.sync_copy(deg_sh.at[pl.ds(out0, 320)], item_deg.at[pl.ds(out0, 320)])

  @pl.when(cid == 1)
  def _():
    pltpu.sync_copy(acc_sh.at[pl.ds(out0, 320)], user_sum.at[pl.ds(out0, 320)])
    pltpu.sync_copy(deg_sh.at[pl.ds(out0, 320)], user_deg.at[pl.ds(out0, 320)])


def _tc_head_body(feat, msum, mdeg, w_t, b, gamma, beta, out):
  d = mdeg[...][:, 0:1]
  r = 1.0 / jnp.maximum(d, 1.0)
  x = feat[...] + msum[...] * r
  h = jnp.dot(x, w_t[...], preferred_element_type=jnp.float32) + b[...]
  h = jnp.maximum(h, 0.0)
  mu = jnp.mean(h, axis=1, keepdims=True)
  var = jnp.mean((h - mu) ** 2, axis=1, keepdims=True)
  out[...] = (h - mu) * lax.rsqrt(var + EPS) * gamma[...] + beta[...]


def _tc_head(feat, msum, mdeg, w_t, b, gamma, beta):
  bm = 1000
  grid = N_NODES // bm
  return pl.pallas_call(
      _tc_head_body,
      grid=(grid,),
      in_specs=[
          pl.BlockSpec((bm, DIM), lambda i: (i, 0)),
          pl.BlockSpec((bm, DIM), lambda i: (i, 0)),
          pl.BlockSpec((bm, DIM), lambda i: (i, 0)),
          pl.BlockSpec((DIM, DIM), lambda i: (0, 0)),
          pl.BlockSpec((1, DIM), lambda i: (0, 0)),
          pl.BlockSpec((1, DIM), lambda i: (0, 0)),
          pl.BlockSpec((1, DIM), lambda i: (0, 0)),
      ],
      out_specs=pl.BlockSpec((bm, DIM), lambda i: (i, 0)),
      out_shape=jax.ShapeDtypeStruct((N_NODES, DIM), jnp.float32),
  )(feat, msum, mdeg, w_t, b, gamma, beta)


@jax.jit
def kernel(user_features, item_features, user_item_edge_index,
           item_user_edge_index, Wu, bu, Wi, bi, gamma, beta):
  ui_src = user_item_edge_index[0].astype(jnp.int32)
  ui_dst = user_item_edge_index[1].astype(jnp.int32)
  iu_src = item_user_edge_index[0].astype(jnp.int32)
  iu_dst = item_user_edge_index[1].astype(jnp.int32)
  zero_f = jnp.zeros((PAD, DIM), jnp.float32)
  ones = jnp.ones((K, DIM), jnp.float32)

  item_sum, item_deg, user_sum, user_deg = _sc_aggregate(
      user_features, item_features, ui_src, ui_dst, iu_src, iu_dst,
      zero_f, ones)

  g = gamma.reshape(1, DIM)
  bt = beta.reshape(1, DIM)
  user_new = _tc_head(user_features, user_sum, user_deg,
                      Wu.T, bu.reshape(1, DIM), g, bt)
  item_new = _tc_head(item_features, item_sum, item_deg,
                      Wi.T, bi.reshape(1, DIM), g, bt)
  return (user_new, item_new)


# speedup vs baseline: 7.0695x; 1.5136x over previous
import functools

import jax
import jax.numpy as jnp
from jax import lax
from jax.experimental import pallas as pl
from jax.experimental.pallas import tpu as pltpu
from jax.experimental.pallas import tpu_sc as plsc

N_NODES = 5000
N_EDGES = 320000
DIM = 128
EPS = 1e-5

NS = 16
PAD = 5120
K = 80
EDGES_PER_TILE = N_EDGES // NS
CHUNKS = EDGES_PER_TILE // K
OUT_STRIDE = 312

_mesh = plsc.VectorSubcoreMesh(core_axis_name="c", subcore_axis_name="s")


@functools.partial(
    pl.kernel,
    out_type=(
        jax.ShapeDtypeStruct((N_NODES, DIM), jnp.float32),
        jax.ShapeDtypeStruct((N_NODES, DIM), jnp.float32),
        jax.ShapeDtypeStruct((N_NODES, DIM), jnp.float32),
        jax.ShapeDtypeStruct((N_NODES, DIM), jnp.float32),
    ),
    mesh=_mesh,
    scratch_types=[
        pltpu.VMEM_SHARED((PAD, DIM), jnp.float32),
        pltpu.VMEM_SHARED((PAD, DIM), jnp.float32),
        pltpu.VMEM((K,), jnp.int32),
        pltpu.VMEM((K,), jnp.int32),
        pltpu.VMEM((K, DIM), jnp.float32),
        pltpu.VMEM((K,), jnp.int32),
        pltpu.VMEM((K,), jnp.int32),
        pltpu.VMEM((K, DIM), jnp.float32),
        pltpu.VMEM((K, DIM), jnp.float32),
        pltpu.SemaphoreType.DMA,
        pltpu.SemaphoreType.DMA,
    ],
)
def _sc_aggregate(uf_hbm, if_hbm, ui_src, ui_dst, iu_src, iu_dst,
                  zero_f, ones_hbm,
                  item_sum, item_deg, user_sum, user_deg,
                  acc_sh, deg_sh, src_a, dst_a, rows_a, src_b, dst_b, rows_b,
                  ones_v, sem_a, sem_b):
  cid = lax.axis_index("c")
  sid = lax.axis_index("s")
  row0 = sid * (PAD // NS)
  edge0 = sid * EDGES_PER_TILE

  pltpu.sync_copy(zero_f.at[pl.ds(row0, PAD // NS)],
                  acc_sh.at[pl.ds(row0, PAD // NS)])
  pltpu.sync_copy(zero_f.at[pl.ds(row0, PAD // NS)],
                  deg_sh.at[pl.ds(row0, PAD // NS)])
  pltpu.sync_copy(ones_hbm, ones_v)
  plsc.subcore_barrier()

  def run(feat_hbm, src_hbm, dst_hbm):
    def load_and_gather(c, src_v, dst_v, rows_v, sem):
      off = edge0 + c * K
      pltpu.sync_copy(src_hbm.at[pl.ds(off, K)], src_v)
      pltpu.sync_copy(dst_hbm.at[pl.ds(off, K)], dst_v)
      pltpu.async_copy(feat_hbm.at[src_v], rows_v, sem)

    def drain_and_scatter(src_v, dst_v, rows_v, sem):
      pltpu.make_async_copy(feat_hbm.at[src_v], rows_v, sem).wait()
      pltpu.sync_copy(rows_v, acc_sh.at[dst_v], add=True)
      pltpu.sync_copy(ones_v, deg_sh.at[dst_v], add=True)

    load_and_gather(0, src_a, dst_a, rows_a, sem_a)
    load_and_gather(1, src_b, dst_b, rows_b, sem_b)

    def step(i, _):
      c = 2 * i
      drain_and_scatter(src_a, dst_a, rows_a, sem_a)
      load_and_gather(c + 2, src_a, dst_a, rows_a, sem_a)
      drain_and_scatter(src_b, dst_b, rows_b, sem_b)
      load_and_gather(c + 3, src_b, dst_b, rows_b, sem_b)
      return ()
    lax.fori_loop(0, CHUNKS // 2 - 1, step, ())

    drain_and_scatter(src_a, dst_a, rows_a, sem_a)
    drain_and_scatter(src_b, dst_b, rows_b, sem_b)

  @pl.when(cid == 0)
  def _():
    run(uf_hbm, ui_src, ui_dst)

  @pl.when(cid == 1)
  def _():
    run(if_hbm, iu_src, iu_dst)

  plsc.subcore_barrier()

  out0 = sid * OUT_STRIDE

  @pl.when(cid == 0)
  def _():
    pltpu.sync_copy(acc_sh.at[pl.ds(out0, 320)], item_sum.at[pl.ds(out0, 320)])
    pltpu.sync_copy(deg_sh.at[pl.ds(out0, 320)], item_deg.at[pl.ds(out0, 320)])

  @pl.when(cid == 1)
  def _():
    pltpu.sync_copy(acc_sh.at[pl.ds(out0, 320)], user_sum.at[pl.ds(out0, 320)])
    pltpu.sync_copy(deg_sh.at[pl.ds(out0, 320)], user_deg.at[pl.ds(out0, 320)])


def _tc_head_body(feat, msum, mdeg, w_t, b, gamma, beta, out):
  d = mdeg[...][:, 0:1]
  r = 1.0 / jnp.maximum(d, 1.0)
  x = feat[...] + msum[...] * r
  h = jnp.dot(x, w_t[...], preferred_element_type=jnp.float32) + b[...]
  h = jnp.maximum(h, 0.0)
  mu = jnp.mean(h, axis=1, keepdims=True)
  var = jnp.mean((h - mu) ** 2, axis=1, keepdims=True)
  out[...] = (h - mu) * lax.rsqrt(var + EPS) * gamma[...] + beta[...]


def _tc_head(feat, msum, mdeg, w_t, b, gamma, beta):
  bm = 1000
  grid = N_NODES // bm
  return pl.pallas_call(
      _tc_head_body,
      grid=(grid,),
      in_specs=[
          pl.BlockSpec((bm, DIM), lambda i: (i, 0)),
          pl.BlockSpec((bm, DIM), lambda i: (i, 0)),
          pl.BlockSpec((bm, DIM), lambda i: (i, 0)),
          pl.BlockSpec((DIM, DIM), lambda i: (0, 0)),
          pl.BlockSpec((1, DIM), lambda i: (0, 0)),
          pl.BlockSpec((1, DIM), lambda i: (0, 0)),
          pl.BlockSpec((1, DIM), lambda i: (0, 0)),
      ],
      out_specs=pl.BlockSpec((bm, DIM), lambda i: (i, 0)),
      out_shape=jax.ShapeDtypeStruct((N_NODES, DIM), jnp.float32),
  )(feat, msum, mdeg, w_t, b, gamma, beta)


@jax.jit
def kernel(user_features, item_features, user_item_edge_index,
           item_user_edge_index, Wu, bu, Wi, bi, gamma, beta):
  ui_src = user_item_edge_index[0].astype(jnp.int32)
  ui_dst = user_item_edge_index[1].astype(jnp.int32)
  iu_src = item_user_edge_index[0].astype(jnp.int32)
  iu_dst = item_user_edge_index[1].astype(jnp.int32)
  zero_f = jnp.zeros((PAD, DIM), jnp.float32)
  ones = jnp.ones((K, DIM), jnp.float32)

  item_sum, item_deg, user_sum, user_deg = _sc_aggregate(
      user_features, item_features, ui_src, ui_dst, iu_src, iu_dst,
      zero_f, ones)

  g = gamma.reshape(1, DIM)
  bt = beta.reshape(1, DIM)
  user_new = _tc_head(user_features, user_sum, user_deg,
                      Wu.T, bu.reshape(1, DIM), g, bt)
  item_new = _tc_head(item_features, item_sum, item_deg,
                      Wi.T, bi.reshape(1, DIM), g, bt)
  return (user_new, item_new)


# speedup vs baseline: 8.7312x; 1.2350x over previous
import functools

import jax
import jax.numpy as jnp
from jax import lax
from jax.experimental import pallas as pl
from jax.experimental.pallas import tpu as pltpu
from jax.experimental.pallas import tpu_sc as plsc

N_NODES = 5000
N_EDGES = 320000
DIM = 128
EPS = 1e-5

NS = 16
PAD = 5120
K = 80
EDGES_PER_TILE = N_EDGES // NS
CHUNKS = EDGES_PER_TILE // K
OUT_STRIDE = 312

_mesh = plsc.VectorSubcoreMesh(core_axis_name="c", subcore_axis_name="s")


@functools.partial(
    pl.kernel,
    out_type=(
        jax.ShapeDtypeStruct((N_NODES, DIM), jnp.float32),
        jax.ShapeDtypeStruct((NS, PAD), jnp.float32),
        jax.ShapeDtypeStruct((N_NODES, DIM), jnp.float32),
        jax.ShapeDtypeStruct((NS, PAD), jnp.float32),
    ),
    mesh=_mesh,
    compiler_params=pltpu.CompilerParams(needs_layout_passes=False),
    scratch_types=[
        pltpu.VMEM_SHARED((PAD, DIM), jnp.float32),
        pltpu.VMEM((K,), jnp.int32),
        pltpu.VMEM((K,), jnp.int32),
        pltpu.VMEM((K, DIM), jnp.float32),
        pltpu.VMEM((K,), jnp.int32),
        pltpu.VMEM((K,), jnp.int32),
        pltpu.VMEM((K, DIM), jnp.float32),
        pltpu.VMEM((PAD,), jnp.float32),
        pltpu.SemaphoreType.DMA,
        pltpu.SemaphoreType.DMA,
    ],
)
def _sc_aggregate(uf_hbm, if_hbm, ui_src, ui_dst, iu_src, iu_dst, zero_f,
                  item_sum, item_deg, user_sum, user_deg,
                  acc_sh, src_a, dst_a, rows_a, src_b, dst_b, rows_b,
                  degl, sem_a, sem_b):
  cid = lax.axis_index("c")
  sid = lax.axis_index("s")
  row0 = sid * (PAD // NS)
  edge0 = sid * EDGES_PER_TILE
  zeros16 = jnp.zeros((16,), jnp.float32)
  ones16 = jnp.ones((16,), jnp.float32)

  pltpu.sync_copy(zero_f.at[pl.ds(row0, PAD // NS)],
                  acc_sh.at[pl.ds(row0, PAD // NS)])

  def zrow(i, _):
    degl[pl.ds(i * 16, 16)] = zeros16
    return ()
  lax.fori_loop(0, PAD // 16, zrow, ())
  plsc.subcore_barrier()

  def run(feat_hbm, src_hbm, dst_hbm, deg_out):
    def load_and_gather(c, src_v, dst_v, rows_v, sem):
      off = edge0 + c * K
      pltpu.sync_copy(src_hbm.at[pl.ds(off, K)], src_v)
      pltpu.sync_copy(dst_hbm.at[pl.ds(off, K)], dst_v)
      pltpu.async_copy(feat_hbm.at[src_v], rows_v, sem)

    def drain_and_scatter(src_v, dst_v, rows_v, sem):
      for i in range(K // 16):
        plsc.addupdate_scatter(degl, [dst_v[pl.ds(i * 16, 16)]], ones16)
      pltpu.make_async_copy(feat_hbm.at[src_v], rows_v, sem).wait()
      pltpu.sync_copy(rows_v, acc_sh.at[dst_v], add=True)

    load_and_gather(0, src_a, dst_a, rows_a, sem_a)
    load_and_gather(1, src_b, dst_b, rows_b, sem_b)

    def step(i, _):
      c = 2 * i
      drain_and_scatter(src_a, dst_a, rows_a, sem_a)
      load_and_gather(c + 2, src_a, dst_a, rows_a, sem_a)
      drain_and_scatter(src_b, dst_b, rows_b, sem_b)
      load_and_gather(c + 3, src_b, dst_b, rows_b, sem_b)
      return ()
    lax.fori_loop(0, CHUNKS // 2 - 1, step, ())

    drain_and_scatter(src_a, dst_a, rows_a, sem_a)
    drain_and_scatter(src_b, dst_b, rows_b, sem_b)

    pltpu.sync_copy(degl, deg_out.at[sid])

  @pl.when(cid == 0)
  def _():
    run(uf_hbm, ui_src, ui_dst, item_deg)

  @pl.when(cid == 1)
  def _():
    run(if_hbm, iu_src, iu_dst, user_deg)

  plsc.subcore_barrier()

  out0 = sid * OUT_STRIDE

  @pl.when(cid == 0)
  def _():
    pltpu.sync_copy(acc_sh.at[pl.ds(out0, 320)], item_sum.at[pl.ds(out0, 320)])

  @pl.when(cid == 1)
  def _():
    pltpu.sync_copy(acc_sh.at[pl.ds(out0, 320)], user_sum.at[pl.ds(out0, 320)])


def _tc_degsum_body(ideg, udeg, ones_c, iout, uout):
  iout[...] = lax.dot_general(ideg[...], ones_c[...], (((0,), (0,)), ((), ())),
                              preferred_element_type=jnp.float32)
  uout[...] = lax.dot_general(udeg[...], ones_c[...], (((0,), (0,)), ((), ())),
                              preferred_element_type=jnp.float32)


def _tc_degsum(item_deg, user_deg):
  ones_c = jnp.ones((NS, 1), jnp.float32)
  return pl.pallas_call(
      _tc_degsum_body,
      out_shape=(jax.ShapeDtypeStruct((PAD, 1), jnp.float32),
                 jax.ShapeDtypeStruct((PAD, 1), jnp.float32)),
  )(item_deg, user_deg, ones_c)


def _tc_head_body(feat, msum, mdeg, w_t, b, gamma, beta, out):
  d = mdeg[...]
  r = 1.0 / jnp.maximum(d, 1.0)
  x = feat[...] + msum[...] * r
  h = jnp.dot(x, w_t[...], preferred_element_type=jnp.float32) + b[...]
  h = jnp.maximum(h, 0.0)
  mu = jnp.mean(h, axis=1, keepdims=True)
  var = jnp.mean((h - mu) ** 2, axis=1, keepdims=True)
  out[...] = (h - mu) * lax.rsqrt(var + EPS) * gamma[...] + beta[...]


def _tc_head(feat, msum, mdeg, w_t, b, gamma, beta):
  bm = 1000
  grid = N_NODES // bm
  return pl.pallas_call(
      _tc_head_body,
      grid=(grid,),
      in_specs=[
          pl.BlockSpec((bm, DIM), lambda i: (i, 0)),
          pl.BlockSpec((bm, DIM), lambda i: (i, 0)),
          pl.BlockSpec((bm, 1), lambda i: (i, 0)),
          pl.BlockSpec((DIM, DIM), lambda i: (0, 0)),
          pl.BlockSpec((1, DIM), lambda i: (0, 0)),
          pl.BlockSpec((1, DIM), lambda i: (0, 0)),
          pl.BlockSpec((1, DIM), lambda i: (0, 0)),
      ],
      out_specs=pl.BlockSpec((bm, DIM), lambda i: (i, 0)),
      out_shape=jax.ShapeDtypeStruct((N_NODES, DIM), jnp.float32),
  )(feat, msum, mdeg, w_t, b, gamma, beta)


@jax.jit
def kernel(user_features, item_features, user_item_edge_index,
           item_user_edge_index, Wu, bu, Wi, bi, gamma, beta):
  ui_src = user_item_edge_index[0].astype(jnp.int32)
  ui_dst = user_item_edge_index[1].astype(jnp.int32)
  iu_src = item_user_edge_index[0].astype(jnp.int32)
  iu_dst = item_user_edge_index[1].astype(jnp.int32)
  zero_f = jnp.zeros((PAD, DIM), jnp.float32)

  item_sum, item_deg, user_sum, user_deg = _sc_aggregate(
      user_features, item_features, ui_src, ui_dst, iu_src, iu_dst, zero_f)

  item_d, user_d = _tc_degsum(item_deg, user_deg)
  g = gamma.reshape(1, DIM)
  bt = beta.reshape(1, DIM)
  user_new = _tc_head(user_features, user_sum, user_d,
                      Wu.T, bu.reshape(1, DIM), g, bt)
  item_new = _tc_head(item_features, item_sum, item_d,
                      Wi.T, bi.reshape(1, DIM), g, bt)
  return (user_new, item_new)


# speedup vs baseline: 9.9710x; 1.1420x over previous
import functools

import jax
import jax.numpy as jnp
from jax import lax
from jax.experimental import pallas as pl
from jax.experimental.pallas import tpu as pltpu
from jax.experimental.pallas import tpu_sc as plsc

N_NODES = 5000
N_EDGES = 320000
DIM = 128
EPS = 1e-5

NS = 16
PAD = 5120
K = 80
EDGES_PER_TILE = N_EDGES // NS
CHUNKS = EDGES_PER_TILE // K
OUT_STRIDE = 312

_mesh = plsc.VectorSubcoreMesh(core_axis_name="c", subcore_axis_name="s")


@functools.partial(
    pl.kernel,
    out_type=(
        jax.ShapeDtypeStruct((N_NODES, DIM), jnp.float32),
        jax.ShapeDtypeStruct((NS, PAD), jnp.float32),
        jax.ShapeDtypeStruct((N_NODES, DIM), jnp.float32),
        jax.ShapeDtypeStruct((NS, PAD), jnp.float32),
    ),
    mesh=_mesh,
    compiler_params=pltpu.CompilerParams(needs_layout_passes=False),
    scratch_types=[
        pltpu.VMEM_SHARED((PAD, DIM), jnp.float32),
    ] + [
        s
        for _ in range(4)
        for s in (pltpu.VMEM((K,), jnp.int32),
                  pltpu.VMEM((K,), jnp.int32),
                  pltpu.VMEM((K, DIM), jnp.float32),
                  pltpu.SemaphoreType.DMA,
                  pltpu.SemaphoreType.DMA,
                  pltpu.SemaphoreType.DMA)
    ] + [
        pltpu.VMEM((PAD,), jnp.float32),
    ],
)
def _sc_aggregate(uf_hbm, if_hbm, ui_src, ui_dst, iu_src, iu_dst, zero_f,
                  item_sum, item_deg, user_sum, user_deg,
                  acc_sh, *rest):
  sets = tuple(rest[6 * x:6 * x + 6] for x in range(4))
  degl = rest[24]
  cid = lax.axis_index("c")
  sid = lax.axis_index("s")
  row0 = sid * (PAD // NS)
  edge0 = sid * EDGES_PER_TILE
  zeros16 = jnp.zeros((16,), jnp.float32)
  ones16 = jnp.ones((16,), jnp.float32)

  pltpu.sync_copy(zero_f.at[pl.ds(row0, PAD // NS)],
                  acc_sh.at[pl.ds(row0, PAD // NS)])

  def zrow(i, _):
    degl[pl.ds(i * 16, 16)] = zeros16
    return ()
  lax.fori_loop(0, PAD // 16, zrow, ())
  plsc.subcore_barrier()

  def run(feat_hbm, src_hbm, dst_hbm, deg_out):
    def fire_idx(c, s):
      off = edge0 + c * K
      pltpu.async_copy(src_hbm.at[pl.ds(off, K)], s[0], s[3])
      pltpu.async_copy(dst_hbm.at[pl.ds(off, K)], s[1], s[3])

    def wait_idx(c, s):
      off = edge0 + c * K
      pltpu.make_async_copy(src_hbm.at[pl.ds(off, K)], s[0], s[3]).wait()
      pltpu.make_async_copy(dst_hbm.at[pl.ds(off, K)], s[1], s[3]).wait()

    def fire_gather(s):
      pltpu.async_copy(feat_hbm.at[s[0]], s[2], s[4])

    def wait_gather(s):
      pltpu.make_async_copy(feat_hbm.at[s[0]], s[2], s[4]).wait()

    def fire_scatter(s):
      pltpu.async_copy(s[2], acc_sh.at[s[1]], s[5], add=True)

    def wait_scatter(s):
      pltpu.make_async_copy(s[2], acc_sh.at[s[1]], s[5]).wait()

    def anchor(c, sk, drain=True, pf_idx=True, pf_gather=True):
      s = sets[sk]
      wait_gather(s)
      for i in range(K // 16):
        plsc.addupdate_scatter(degl, [s[1][pl.ds(i * 16, 16)]], ones16)
      fire_scatter(s)
      if pf_idx:
        y = sets[(sk + 2) % 4]
        if drain:
          wait_scatter(y)
        fire_idx(c + 2, y)
      if pf_gather:
        z = sets[(sk + 1) % 4]
        wait_idx(c + 1, z)
        fire_gather(z)

    fire_idx(0, sets[0])
    fire_idx(1, sets[1])
    wait_idx(0, sets[0])
    fire_gather(sets[0])
    anchor(0, 0, drain=False)
    anchor(1, 1, drain=False)

    def step(i, _):
      c0 = 4 * i + 2
      for k in range(4):
        anchor(c0 + k, (2 + k) % 4)
      return ()
    lax.fori_loop(0, (CHUNKS - 6) // 4, step, ())

    anchor(CHUNKS - 4, (CHUNKS - 4) % 4)
    anchor(CHUNKS - 3, (CHUNKS - 3) % 4)
    anchor(CHUNKS - 2, (CHUNKS - 2) % 4, pf_idx=False)
    anchor(CHUNKS - 1, (CHUNKS - 1) % 4, pf_idx=False, pf_gather=False)
    for k in range(4):
      wait_scatter(sets[(CHUNKS - 4 + k) % 4])

    pltpu.sync_copy(degl, deg_out.at[sid])

  @pl.when(cid == 0)
  def _():
    run(uf_hbm, ui_src, ui_dst, item_deg)

  @pl.when(cid == 1)
  def _():
    run(if_hbm, iu_src, iu_dst, user_deg)

  plsc.subcore_barrier()

  out0 = sid * OUT_STRIDE

  @pl.when(cid == 0)
  def _():
    pltpu.sync_copy(acc_sh.at[pl.ds(out0, 320)], item_sum.at[pl.ds(out0, 320)])

  @pl.when(cid == 1)
  def _():
    pltpu.sync_copy(acc_sh.at[pl.ds(out0, 320)], user_sum.at[pl.ds(out0, 320)])


def _tc_degsum_body(ideg, udeg, ones_c, iout, uout):
  iout[...] = lax.dot_general(ideg[...], ones_c[...], (((0,), (0,)), ((), ())),
                              preferred_element_type=jnp.float32)
  uout[...] = lax.dot_general(udeg[...], ones_c[...], (((0,), (0,)), ((), ())),
                              preferred_element_type=jnp.float32)


def _tc_degsum(item_deg, user_deg):
  ones_c = jnp.ones((NS, 1), jnp.float32)
  return pl.pallas_call(
      _tc_degsum_body,
      out_shape=(jax.ShapeDtypeStruct((PAD, 1), jnp.float32),
                 jax.ShapeDtypeStruct((PAD, 1), jnp.float32)),
  )(item_deg, user_deg, ones_c)


def _tc_head_body(feat, msum, mdeg, w_t, b, gamma, beta, out):
  d = mdeg[...]
  r = 1.0 / jnp.maximum(d, 1.0)
  x = feat[...] + msum[...] * r
  h = jnp.dot(x, w_t[...], preferred_element_type=jnp.float32) + b[...]
  h = jnp.maximum(h, 0.0)
  mu = jnp.mean(h, axis=1, keepdims=True)
  var = jnp.mean((h - mu) ** 2, axis=1, keepdims=True)
  out[...] = (h - mu) * lax.rsqrt(var + EPS) * gamma[...] + beta[...]


def _tc_head(feat, msum, mdeg, w_t, b, gamma, beta):
  bm = 1000
  grid = N_NODES // bm
  return pl.pallas_call(
      _tc_head_body,
      grid=(grid,),
      in_specs=[
          pl.BlockSpec((bm, DIM), lambda i: (i, 0)),
          pl.BlockSpec((bm, DIM), lambda i: (i, 0)),
          pl.BlockSpec((bm, 1), lambda i: (i, 0)),
          pl.BlockSpec((DIM, DIM), lambda i: (0, 0)),
          pl.BlockSpec((1, DIM), lambda i: (0, 0)),
          pl.BlockSpec((1, DIM), lambda i: (0, 0)),
          pl.BlockSpec((1, DIM), lambda i: (0, 0)),
      ],
      out_specs=pl.BlockSpec((bm, DIM), lambda i: (i, 0)),
      out_shape=jax.ShapeDtypeStruct((N_NODES, DIM), jnp.float32),
  )(feat, msum, mdeg, w_t, b, gamma, beta)


@jax.jit
def kernel(user_features, item_features, user_item_edge_index,
           item_user_edge_index, Wu, bu, Wi, bi, gamma, beta):
  ui_src = user_item_edge_index[0].astype(jnp.int32)
  ui_dst = user_item_edge_index[1].astype(jnp.int32)
  iu_src = item_user_edge_index[0].astype(jnp.int32)
  iu_dst = item_user_edge_index[1].astype(jnp.int32)
  zero_f = jnp.zeros((PAD, DIM), jnp.float32)

  item_sum, item_deg, user_sum, user_deg = _sc_aggregate(
      user_features, item_features, ui_src, ui_dst, iu_src, iu_dst, zero_f)

  item_d, user_d = _tc_degsum(item_deg, user_deg)
  g = gamma.reshape(1, DIM)
  bt = beta.reshape(1, DIM)
  user_new = _tc_head(user_features, user_sum, user_d,
                      Wu.T, bu.reshape(1, DIM), g, bt)
  item_new = _tc_head(item_features, item_sum, item_d,
                      Wi.T, bi.reshape(1, DIM), g, bt)
  return (user_new, item_new)


# speedup vs baseline: 10.1168x; 1.0146x over previous
import functools

import jax
import jax.numpy as jnp
from jax import lax
from jax.experimental import pallas as pl
from jax.experimental.pallas import tpu as pltpu
from jax.experimental.pallas import tpu_sc as plsc

N_NODES = 5000
N_EDGES = 320000
DIM = 128
EPS = 1e-5

NS = 16
PAD = 5120
K = 80
EDGES_PER_TILE = N_EDGES // NS
CHUNKS = EDGES_PER_TILE // K
OUT_STRIDE = 312

_mesh = plsc.VectorSubcoreMesh(core_axis_name="c", subcore_axis_name="s")


@functools.partial(
    pl.kernel,
    out_type=(
        jax.ShapeDtypeStruct((N_NODES, DIM), jnp.float32),
        jax.ShapeDtypeStruct((NS, PAD), jnp.float32),
        jax.ShapeDtypeStruct((N_NODES, DIM), jnp.float32),
        jax.ShapeDtypeStruct((NS, PAD), jnp.float32),
    ),
    mesh=_mesh,
    compiler_params=pltpu.CompilerParams(needs_layout_passes=False),
    scratch_types=[
        pltpu.VMEM_SHARED((PAD, DIM), jnp.float32),
    ] + [
        s
        for _ in range(4)
        for s in (pltpu.VMEM((K,), jnp.int32),
                  pltpu.VMEM((K,), jnp.int32),
                  pltpu.VMEM((K, DIM), jnp.float32),
                  pltpu.SemaphoreType.DMA,
                  pltpu.SemaphoreType.DMA,
                  pltpu.SemaphoreType.DMA)
    ] + [
        pltpu.VMEM((PAD,), jnp.float32),
    ],
)
def _sc_aggregate(uf_hbm, if_hbm, ui_src, ui_dst, iu_src, iu_dst, zero_f,
                  item_sum, item_deg, user_sum, user_deg,
                  acc_sh, *rest):
  sets = tuple(rest[6 * x:6 * x + 6] for x in range(4))
  degl = rest[24]
  cid = lax.axis_index("c")
  sid = lax.axis_index("s")
  row0 = sid * (PAD // NS)
  edge0 = sid * EDGES_PER_TILE
  zeros16 = jnp.zeros((16,), jnp.float32)
  ones16 = jnp.ones((16,), jnp.float32)

  pltpu.sync_copy(zero_f, acc_sh.at[pl.ds(row0, PAD // NS)])

  def zrow(i, _):
    degl[pl.ds(i * 16, 16)] = zeros16
    return ()
  lax.fori_loop(0, PAD // 16, zrow, ())
  plsc.subcore_barrier()

  def run(feat_hbm, src_hbm, dst_hbm, deg_out):
    def fire_idx(c, s):
      off = edge0 + c * K
      pltpu.async_copy(src_hbm.at[pl.ds(off, K)], s[0], s[3])
      pltpu.async_copy(dst_hbm.at[pl.ds(off, K)], s[1], s[3])

    def wait_idx(c, s):
      off = edge0 + c * K
      pltpu.make_async_copy(src_hbm.at[pl.ds(off, K)], s[0], s[3]).wait()
      pltpu.make_async_copy(dst_hbm.at[pl.ds(off, K)], s[1], s[3]).wait()

    def fire_gather(s):
      pltpu.async_copy(feat_hbm.at[s[0]], s[2], s[4])

    def wait_gather(s):
      pltpu.make_async_copy(feat_hbm.at[s[0]], s[2], s[4]).wait()

    def fire_scatter(s):
      pltpu.async_copy(s[2], acc_sh.at[s[1]], s[5], add=True)

    def wait_scatter(s):
      pltpu.make_async_copy(s[2], acc_sh.at[s[1]], s[5]).wait()

    def anchor(c, sk, drain=True, pf_idx=True, pf_gather=True):
      s = sets[sk]
      wait_gather(s)
      for i in range(K // 16):
        plsc.addupdate_scatter(degl, [s[1][pl.ds(i * 16, 16)]], ones16)
      fire_scatter(s)
      if pf_idx:
        y = sets[(sk + 2) % 4]
        if drain:
          wait_scatter(y)
        fire_idx(c + 2, y)
      if pf_gather:
        z = sets[(sk + 1) % 4]
        wait_idx(c + 1, z)
        fire_gather(z)

    fire_idx(0, sets[0])
    fire_idx(1, sets[1])
    wait_idx(0, sets[0])
    fire_gather(sets[0])
    anchor(0, 0, drain=False)
    anchor(1, 1, drain=False)

    def step(i, _):
      c0 = 4 * i + 2
      for k in range(4):
        anchor(c0 + k, (2 + k) % 4)
      return ()
    lax.fori_loop(0, (CHUNKS - 6) // 4, step, ())

    anchor(CHUNKS - 4, (CHUNKS - 4) % 4)
    anchor(CHUNKS - 3, (CHUNKS - 3) % 4)
    anchor(CHUNKS - 2, (CHUNKS - 2) % 4, pf_idx=False)
    anchor(CHUNKS - 1, (CHUNKS - 1) % 4, pf_idx=False, pf_gather=False)
    for k in range(4):
      wait_scatter(sets[(CHUNKS - 4 + k) % 4])

    pltpu.sync_copy(degl, deg_out.at[sid])

  @pl.when(cid == 0)
  def _():
    run(uf_hbm, ui_src, ui_dst, item_deg)

  @pl.when(cid == 1)
  def _():
    run(if_hbm, iu_src, iu_dst, user_deg)

  plsc.subcore_barrier()

  out0 = sid * OUT_STRIDE

  @pl.when(cid == 0)
  def _():
    pltpu.sync_copy(acc_sh.at[pl.ds(out0, 320)], item_sum.at[pl.ds(out0, 320)])

  @pl.when(cid == 1)
  def _():
    pltpu.sync_copy(acc_sh.at[pl.ds(out0, 320)], user_sum.at[pl.ds(out0, 320)])


def _tc_head_body(feat, msum, mdeg, ones_c, w_t, b, gamma, beta, out):
  d = lax.dot_general(mdeg[...], ones_c[...], (((0,), (0,)), ((), ())),
                      preferred_element_type=jnp.float32)
  r = 1.0 / jnp.maximum(d, 1.0)
  x = feat[...] + msum[...] * r
  h = jnp.dot(x, w_t[...], preferred_element_type=jnp.float32) + b[...]
  h = jnp.maximum(h, 0.0)
  mu = jnp.mean(h, axis=1, keepdims=True)
  var = jnp.mean((h - mu) ** 2, axis=1, keepdims=True)
  out[...] = (h - mu) * lax.rsqrt(var + EPS) * gamma[...] + beta[...]


def _tc_head(feat, msum, mdeg, ones_c, w_t, b, gamma, beta):
  bm = 1024
  grid = PAD // bm
  return pl.pallas_call(
      _tc_head_body,
      grid=(grid,),
      in_specs=[
          pl.BlockSpec((bm, DIM), lambda i: (i, 0)),
          pl.BlockSpec((bm, DIM), lambda i: (i, 0)),
          pl.BlockSpec((NS, bm), lambda i: (0, i)),
          pl.BlockSpec((NS, 1), lambda i: (0, 0)),
          pl.BlockSpec((DIM, DIM), lambda i: (0, 0)),
          pl.BlockSpec((1, DIM), lambda i: (0, 0)),
          pl.BlockSpec((1, DIM), lambda i: (0, 0)),
          pl.BlockSpec((1, DIM), lambda i: (0, 0)),
      ],
      out_specs=pl.BlockSpec((bm, DIM), lambda i: (i, 0)),
      out_shape=jax.ShapeDtypeStruct((N_NODES, DIM), jnp.float32),
  )(feat, msum, mdeg, ones_c, w_t, b, gamma, beta)


@jax.jit
def kernel(user_features, item_features, user_item_edge_index,
           item_user_edge_index, Wu, bu, Wi, bi, gamma, beta):
  ui_src = user_item_edge_index[0].astype(jnp.int32)
  ui_dst = user_item_edge_index[1].astype(jnp.int32)
  iu_src = item_user_edge_index[0].astype(jnp.int32)
  iu_dst = item_user_edge_index[1].astype(jnp.int32)
  zero_f = jnp.zeros((PAD // NS, DIM), jnp.float32)

  item_sum, item_deg, user_sum, user_deg = _sc_aggregate(
      user_features, item_features, ui_src, ui_dst, iu_src, iu_dst, zero_f)

  ones_c = jnp.ones((NS, 1), jnp.float32)
  g = gamma.reshape(1, DIM)
  bt = beta.reshape(1, DIM)
  user_new = _tc_head(user_features, user_sum, user_deg, ones_c,
                      Wu.T, bu.reshape(1, DIM), g, bt)
  item_new = _tc_head(item_features, item_sum, item_deg, ones_c,
                      Wi.T, bi.reshape(1, DIM), g, bt)
  return (user_new, item_new)
